# baseline (speedup 1.0000x reference)
import functools

import jax
import jax.numpy as jnp
from jax.experimental import pallas as pl
from jax.experimental.pallas import tpu as pltpu

B, S, DM, QIN, H, D, TOPK = 2, 2048, 2048, 1536, 8, 128, 1024
SCALE = D ** -0.5
_PREC = jax.lax.Precision.DEFAULT

TQP = 512
TQS = 128
LOGN = 11


def _proj_body(qin_ref, x_ref, wq_ref, wk_ref, g_ref, b_ref, ww_ref,
               q_ref, k_ref, w_ref):
    qin = qin_ref[...]
    x = x_ref[...]
    q_ref[...] = jax.lax.dot_general(
        qin, wq_ref[...], (((1,), (1,)), ((), ())), precision=_PREC)
    y = jax.lax.dot_general(
        x, wk_ref[...], (((1,), (1,)), ((), ())), precision=_PREC)
    mu = jnp.mean(y, axis=-1, keepdims=True)
    var = jnp.mean((y - mu) ** 2, axis=-1, keepdims=True)
    k_ref[...] = (y - mu) / jnp.sqrt(var + 1e-5) * g_ref[...] + b_ref[...]
    w = jax.lax.dot_general(
        x, ww_ref[...], (((1,), (1,)), ((), ())), precision=_PREC)
    w_ref[...] = w * (H ** -0.5)


def _scores_body(q_ref, k_ref, w_ref, i_ref):
    k = k_ref[0]
    w = w_ref[0]
    acc = jnp.zeros((TQS, S), jnp.float32)
    for h in range(H):
        qh = q_ref[0, :, h * D:(h + 1) * D]
        sh = jax.lax.dot_general(
            qh, k, (((1,), (1,)), ((), ())), precision=_PREC) * SCALE
        acc = acc + sh * w[:, h:h + 1]

    lane = jax.lax.broadcasted_iota(jnp.int32, (TQS, S), 1)

    def mk_pass(m, n, lane_n):
        def body(j, carry):
            v, idx = carry
            d = jnp.int32(m >> 1) >> j
            is_lo = (lane_n & d) == 0
            pv = jnp.where(is_lo, pltpu.roll(v, n - d, 1), pltpu.roll(v, d, 1))
            pi = jnp.where(is_lo, pltpu.roll(idx, n - d, 1), pltpu.roll(idx, d, 1))
            win = (pv > v) | ((pv == v) & (pi < idx))
            desc = (lane_n & m) == 0
            take = win ^ (~is_lo) ^ (~desc)
            return jnp.where(take, pv, v), jnp.where(take, pi, idx)
        return body

    carry = (acc, lane)
    for s in range(1, LOGN):
        carry = jax.lax.fori_loop(0, s, mk_pass(1 << s, S, lane), carry)

    v, idx = carry
    d = S // 2
    is_lo = (lane & d) == 0
    pv = jnp.where(is_lo, pltpu.roll(v, S - d, 1), pltpu.roll(v, d, 1))
    pi = jnp.where(is_lo, pltpu.roll(idx, S - d, 1), pltpu.roll(idx, d, 1))
    win = (pv > v) | ((pv == v) & (pi < idx))
    take = win ^ (~is_lo)
    v = jnp.where(take, pv, v)[:, :TOPK]
    idx = jnp.where(take, pi, idx)[:, :TOPK]

    lane_h = jax.lax.broadcasted_iota(jnp.int32, (TQS, TOPK), 1)
    _, idx = jax.lax.fori_loop(
        0, LOGN - 1, mk_pass(1 << (LOGN - 1), TOPK, lane_h), (v, idx))
    i_ref[0] = idx


def kernel(x, q_input, Wq, Wk, gamma, beta, Ww):
    xf = x.reshape(B * S, DM)
    qf = q_input.reshape(B * S, QIN)

    y = x @ Wk.T
    mu = jnp.mean(y, axis=-1, keepdims=True)
    var = jnp.var(y, axis=-1, keepdims=True)
    k3 = (y - mu) / jnp.sqrt(var + 1e-5) * gamma + beta

    n_p = (B * S) // TQP
    q, _, w = pl.pallas_call(
        _proj_body,
        grid=(n_p,),
        in_specs=[
            pl.BlockSpec((TQP, QIN), lambda i: (i, 0)),
            pl.BlockSpec((TQP, DM), lambda i: (i, 0)),
            pl.BlockSpec((H * D, QIN), lambda i: (0, 0)),
            pl.BlockSpec((D, DM), lambda i: (0, 0)),
            pl.BlockSpec((1, D), lambda i: (0, 0)),
            pl.BlockSpec((1, D), lambda i: (0, 0)),
            pl.BlockSpec((H, DM), lambda i: (0, 0)),
        ],
        out_specs=[
            pl.BlockSpec((TQP, H * D), lambda i: (i, 0)),
            pl.BlockSpec((TQP, D), lambda i: (i, 0)),
            pl.BlockSpec((TQP, H), lambda i: (i, 0)),
        ],
        out_shape=[
            jax.ShapeDtypeStruct((B * S, H * D), jnp.float32),
            jax.ShapeDtypeStruct((B * S, D), jnp.float32),
            jax.ShapeDtypeStruct((B * S, H), jnp.float32),
        ],
    )(qf, xf, Wq, Wk, gamma.reshape(1, D), beta.reshape(1, D), Ww)

    q3 = q.reshape(B, S, H * D)
    w3 = w.reshape(B, S, H)

    idx = pl.pallas_call(
        _scores_body,
        grid=(B, S // TQS),
        in_specs=[
            pl.BlockSpec((1, TQS, H * D), lambda b, i: (b, i, 0)),
            pl.BlockSpec((1, S, D), lambda b, i: (b, 0, 0)),
            pl.BlockSpec((1, TQS, H), lambda b, i: (b, i, 0)),
        ],
        out_specs=pl.BlockSpec((1, TQS, TOPK), lambda b, i: (b, i, 0)),
        out_shape=jax.ShapeDtypeStruct((B, S, TOPK), jnp.int32),
    )(q3, k3, w3)
    return idx


# speedup vs baseline: 3.8134x; 3.8134x over previous
import functools

import jax
import jax.numpy as jnp
from jax.experimental import pallas as pl
from jax.experimental.pallas import tpu as pltpu

B, S, DM, QIN, H, D, TOPK = 2, 2048, 2048, 1536, 8, 128, 1024
SCALE = D ** -0.5
_PREC = jax.lax.Precision.DEFAULT

TQP = 512
TQS = 128
LOGN = 11


def _proj_body(qin_ref, x_ref, wq_ref, wk_ref, g_ref, b_ref, ww_ref,
               q_ref, k_ref, w_ref):
    qin = qin_ref[...]
    x = x_ref[...]
    q_ref[...] = jax.lax.dot_general(
        qin, wq_ref[...], (((1,), (1,)), ((), ())), precision=_PREC)
    y = jax.lax.dot_general(
        x, wk_ref[...], (((1,), (1,)), ((), ())), precision=_PREC)
    mu = jnp.mean(y, axis=-1, keepdims=True)
    var = jnp.mean((y - mu) ** 2, axis=-1, keepdims=True)
    k_ref[...] = (y - mu) / jnp.sqrt(var + 1e-5) * g_ref[...] + b_ref[...]
    w = jax.lax.dot_general(
        x, ww_ref[...], (((1,), (1,)), ((), ())), precision=_PREC)
    w_ref[...] = w * (H ** -0.5)


def _scores_only_body(q_ref, k_ref, w_ref, s_ref):
    k = k_ref[0]
    w = w_ref[0]
    acc = jnp.zeros((TQS, S), jnp.float32)
    for h in range(H):
        qh = q_ref[0, :, h * D:(h + 1) * D]
        sh = jax.lax.dot_general(
            qh, k, (((1,), (1,)), ((), ())), precision=_PREC) * SCALE
        acc = acc + sh * w[:, h:h + 1]
    s_ref[0] = acc


def _scores_body(q_ref, k_ref, w_ref, i_ref):
    k = k_ref[0]
    w = w_ref[0]
    acc = jnp.zeros((TQS, S), jnp.float32)
    for h in range(H):
        qh = q_ref[0, :, h * D:(h + 1) * D]
        sh = jax.lax.dot_general(
            qh, k, (((1,), (1,)), ((), ())), precision=_PREC) * SCALE
        acc = acc + sh * w[:, h:h + 1]

    lane = jax.lax.broadcasted_iota(jnp.int32, (TQS, S), 1)

    def mk_pass(m, n, lane_n):
        def body(j, carry):
            v, idx = carry
            d = jnp.int32(m >> 1) >> j
            is_lo = (lane_n & d) == 0
            pv = jnp.where(is_lo, pltpu.roll(v, n - d, 1), pltpu.roll(v, d, 1))
            pi = jnp.where(is_lo, pltpu.roll(idx, n - d, 1), pltpu.roll(idx, d, 1))
            win = (pv > v) | ((pv == v) & (pi < idx))
            desc = (lane_n & m) == 0
            take = win ^ (~is_lo) ^ (~desc)
            return jnp.where(take, pv, v), jnp.where(take, pi, idx)
        return body

    carry = (acc, lane)
    for s in range(1, LOGN):
        carry = jax.lax.fori_loop(0, s, mk_pass(1 << s, S, lane), carry)

    v, idx = carry
    d = S // 2
    is_lo = (lane & d) == 0
    pv = jnp.where(is_lo, pltpu.roll(v, S - d, 1), pltpu.roll(v, d, 1))
    pi = jnp.where(is_lo, pltpu.roll(idx, S - d, 1), pltpu.roll(idx, d, 1))
    win = (pv > v) | ((pv == v) & (pi < idx))
    take = win ^ (~is_lo)
    v = jnp.where(take, pv, v)[:, :TOPK]
    idx = jnp.where(take, pi, idx)[:, :TOPK]

    lane_h = jax.lax.broadcasted_iota(jnp.int32, (TQS, TOPK), 1)
    _, idx = jax.lax.fori_loop(
        0, LOGN - 1, mk_pass(1 << (LOGN - 1), TOPK, lane_h), (v, idx))
    i_ref[0] = idx


def kernel(x, q_input, Wq, Wk, gamma, beta, Ww):
    xf = x.reshape(B * S, DM)
    qf = q_input.reshape(B * S, QIN)

    y = x @ Wk.T
    mu = jnp.mean(y, axis=-1, keepdims=True)
    var = jnp.var(y, axis=-1, keepdims=True)
    k3 = (y - mu) / jnp.sqrt(var + 1e-5) * gamma + beta

    n_p = (B * S) // TQP
    q, _, w = pl.pallas_call(
        _proj_body,
        grid=(n_p,),
        in_specs=[
            pl.BlockSpec((TQP, QIN), lambda i: (i, 0)),
            pl.BlockSpec((TQP, DM), lambda i: (i, 0)),
            pl.BlockSpec((H * D, QIN), lambda i: (0, 0)),
            pl.BlockSpec((D, DM), lambda i: (0, 0)),
            pl.BlockSpec((1, D), lambda i: (0, 0)),
            pl.BlockSpec((1, D), lambda i: (0, 0)),
            pl.BlockSpec((H, DM), lambda i: (0, 0)),
        ],
        out_specs=[
            pl.BlockSpec((TQP, H * D), lambda i: (i, 0)),
            pl.BlockSpec((TQP, D), lambda i: (i, 0)),
            pl.BlockSpec((TQP, H), lambda i: (i, 0)),
        ],
        out_shape=[
            jax.ShapeDtypeStruct((B * S, H * D), jnp.float32),
            jax.ShapeDtypeStruct((B * S, D), jnp.float32),
            jax.ShapeDtypeStruct((B * S, H), jnp.float32),
        ],
    )(qf, xf, Wq, Wk, gamma.reshape(1, D), beta.reshape(1, D), Ww)

    q3 = q.reshape(B, S, H * D)
    w3 = w.reshape(B, S, H)

    scores = pl.pallas_call(
        _scores_only_body,
        grid=(B, S // TQS),
        in_specs=[
            pl.BlockSpec((1, TQS, H * D), lambda b, i: (b, i, 0)),
            pl.BlockSpec((1, S, D), lambda b, i: (b, 0, 0)),
            pl.BlockSpec((1, TQS, H), lambda b, i: (b, i, 0)),
        ],
        out_specs=pl.BlockSpec((1, TQS, S), lambda b, i: (b, i, 0)),
        out_shape=jax.ShapeDtypeStruct((B, S, S), jnp.float32),
    )(q3, k3, w3)
    _, idx = jax.lax.top_k(scores, TOPK)
    return idx
